# baseline (speedup 1.0000x reference)
import functools

import jax
import jax.numpy as jnp
from jax import lax
from jax.experimental import pallas as pl
from jax.experimental.pallas import tpu as pltpu
from jax.experimental.pallas import tpu_sc as plsc

D = 64
B = 1024

NC = 2
NS = 16
NW = NC * NS
BPW = B // NW
L = 16

_mesh = plsc.VectorSubcoreMesh(core_axis_name="c", subcore_axis_name="s")


@functools.partial(
    pl.kernel,
    mesh=_mesh,
    compiler_params=pltpu.CompilerParams(use_tc_tiling_on_sc=False),
    out_type=[
        jax.ShapeDtypeStruct((B, D), jnp.float32),
        jax.ShapeDtypeStruct((B, D), jnp.float32),
        jax.ShapeDtypeStruct((B, D), jnp.float32),
        jax.ShapeDtypeStruct((B,), jnp.float32),
        jax.ShapeDtypeStruct((B,), jnp.float32),
    ],
    scratch_types=[
        pltpu.VMEM((BPW,), jnp.int32),
        pltpu.VMEM((BPW,), jnp.int32),
        pltpu.VMEM((BPW, D), jnp.float32),
        pltpu.VMEM((BPW, D), jnp.float32),
        pltpu.VMEM((BPW, D), jnp.float32),
        pltpu.VMEM((BPW,), jnp.float32),
        pltpu.VMEM((BPW,), jnp.float32),
        pltpu.VMEM((BPW,), jnp.float32),
        pltpu.VMEM((BPW,), jnp.float32),
        pltpu.VMEM((BPW,), jnp.float32),
        pltpu.VMEM((BPW,), jnp.float32),
        pltpu.SemaphoreType.DMA,
    ],
)
def _sc_gather(uid_hbm, iid_hbm, ue_t, ie_t, cie_t, ub_t, ib_t, cub_t, cib_t,
               ue_o, ie_o, cie_o, bc_o, br_o,
               uidx_v, iidx_v, ue_v, ie_v, cie_v,
               ub_v, ib_v, cub_v, cib_v, bc_v, br_v, sem):
    wid = lax.axis_index("s") * NC + lax.axis_index("c")
    base = wid * BPW
    pltpu.sync_copy(uid_hbm.at[pl.ds(base, BPW)], uidx_v)
    pltpu.sync_copy(iid_hbm.at[pl.ds(base, BPW)], iidx_v)
    copies = [
        pltpu.async_copy(ue_t.at[uidx_v], ue_v, sem),
        pltpu.async_copy(ie_t.at[iidx_v], ie_v, sem),
        pltpu.async_copy(cie_t.at[iidx_v], cie_v, sem),
        pltpu.async_copy(ub_t.at[uidx_v], ub_v, sem),
        pltpu.async_copy(ib_t.at[iidx_v], ib_v, sem),
        pltpu.async_copy(cub_t.at[uidx_v], cub_v, sem),
        pltpu.async_copy(cib_t.at[iidx_v], cib_v, sem),
    ]
    for c in copies:
        c.wait()
    for h in range(BPW // L):
        sl = pl.ds(L * h, L)
        bc_v[sl] = cub_v[sl] + cib_v[sl]
        br_v[sl] = ub_v[sl] + ib_v[sl]
    pltpu.sync_copy(ue_v, ue_o.at[pl.ds(base, BPW)])
    pltpu.sync_copy(ie_v, ie_o.at[pl.ds(base, BPW)])
    pltpu.sync_copy(cie_v, cie_o.at[pl.ds(base, BPW)])
    pltpu.sync_copy(bc_v, bc_o.at[pl.ds(base, BPW)])
    pltpu.sync_copy(br_v, br_o.at[pl.ds(base, BPW)])


RB = 128


def _tc_body(ue_ref, ie_ref, cie_ref, bc_ref, br_ref, o_ref):
    ue = ue_ref[...]
    ones = jnp.ones((1, D), dtype=jnp.float32)
    cd = lax.dot_general(ones, ue * cie_ref[...], (((1,), (1,)), ((), ())),
                         precision=lax.Precision.HIGHEST,
                         preferred_element_type=jnp.float32)
    rd = lax.dot_general(ones, ue * ie_ref[...], (((1,), (1,)), ((), ())),
                         precision=lax.Precision.HIGHEST,
                         preferred_element_type=jnp.float32)
    obs = 1.0 / (1.0 + jnp.exp(-(cd + bc_ref[...])))
    o_ref[...] = obs * (rd + br_ref[...])


_tc_map = pl.pallas_call(
    _tc_body,
    grid=(B // RB,),
    in_specs=[
        pl.BlockSpec((B, D), lambda i: (0, 0)),
        pl.BlockSpec((B, D), lambda i: (0, 0)),
        pl.BlockSpec((B, D), lambda i: (0, 0)),
        pl.BlockSpec((RB, 1), lambda i: (i, 0)),
        pl.BlockSpec((RB, 1), lambda i: (i, 0)),
    ],
    out_specs=pl.BlockSpec((RB, B), lambda i: (i, 0)),
    out_shape=jax.ShapeDtypeStruct((B, B), jnp.float32),
)


def kernel(user_ids, item_ids, user_emb, item_emb, cens_item_emb,
           user_bias, item_bias, cens_user_bias, cens_item_bias):
    uid = user_ids.astype(jnp.int32)
    iid = item_ids.astype(jnp.int32)
    ue, ie, cie, bc, br = _sc_gather(
        uid, iid, user_emb, item_emb, cens_item_emb,
        user_bias.reshape(-1), item_bias.reshape(-1),
        cens_user_bias.reshape(-1), cens_item_bias.reshape(-1))
    return _tc_map(ue, ie, cie, bc.reshape(B, 1), br.reshape(B, 1))
